# Initial kernel scaffold; baseline (speedup 1.0000x reference)
import jax
import jax.numpy as jnp
from jax.experimental import pallas as pl


def kernel(indices):
    raise NotImplementedError("write your pallas kernel here")


# speedup vs baseline: 4.0841x; 4.0841x over previous
import jax
import jax.numpy as jnp
from jax.experimental import pallas as pl

_NUM_CATEGORIES = 256
_BATCH_BLOCK = 16


def _one_hot_block(idx_ref, out_ref):
    idx = idx_ref[...]
    cat = jax.lax.broadcasted_iota(
        jnp.int32, (idx.shape[0], _NUM_CATEGORIES, idx.shape[1]), 1)
    out_ref[...] = (idx[:, None, :] == cat).astype(jnp.float32)


def kernel(indices):
    batch, seq = indices.shape
    bblk = _BATCH_BLOCK
    grid = (batch // bblk,)
    return pl.pallas_call(
        _one_hot_block,
        grid=grid,
        in_specs=[pl.BlockSpec((bblk, seq), lambda i: (i, 0))],
        out_specs=pl.BlockSpec((bblk, _NUM_CATEGORIES, seq), lambda i: (i, 0, 0)),
        out_shape=jax.ShapeDtypeStruct((batch, _NUM_CATEGORIES, seq), jnp.float32),
    )(indices)
